# baseline (speedup 1.0000x reference)
import functools

import jax
import jax.numpy as jnp
from jax.experimental import pallas as pl
from jax.experimental.pallas import tpu as pltpu
from jax.experimental.pallas import tpu_sc as plsc

_NE = 8192
_D = 256
_M = 8192
_BM = 512
_BN = 2048
_BIG = 2**30
_COMMITMENT = 0.25


def _cn_kernel(e_ref, cn_ref):
    e = e_ref[...]
    cn_ref[...] = jnp.sum(e * e, axis=1)[None, :]


def _dist_argmin_kernel(z_ref, e_ref, cn_ref, idx_ref, rn_s, best_s, bidx_s):
    j = pl.program_id(0)
    i = pl.program_id(1)
    z = z_ref[...]
    row = pl.ds(i * _BM, _BM)

    @pl.when(j == 0)
    def _():
        rn_s[row, :] = jnp.sum(z * z, axis=1, keepdims=True)
        best_s[row, :] = jnp.full((_BM, 1), jnp.inf, jnp.float32)
        bidx_s[row, :] = jnp.full((_BM, 1), _BIG, jnp.int32)

    @pl.when(j == pl.num_programs(0) // 2)
    def _():
        best_s[row, :] = best_s[row, :].astype(jnp.bfloat16).astype(jnp.float32)

    e = e_ref[...]
    mm = jax.lax.dot_general(
        z.astype(jnp.bfloat16), e.astype(jnp.bfloat16),
        (((1,), (1,)), ((), ())),
        preferred_element_type=jnp.float32)
    d = (rn_s[row, :] + cn_ref[...]) - 2.0 * mm
    m = jnp.min(d, axis=1, keepdims=True)
    iota = jax.lax.broadcasted_iota(jnp.int32, d.shape, 1) + j * _BN
    i_j = jnp.min(jnp.where(d == m, iota, _BIG), axis=1, keepdims=True)
    upd = m < best_s[row, :]
    bidx_s[row, :] = jnp.where(upd, i_j, bidx_s[row, :])
    best_s[row, :] = jnp.where(upd, m, best_s[row, :])
    idx_ref[...] = bidx_s[row, :]


def _argmin_codes(flat, embedding):
    cn = pl.pallas_call(
        _cn_kernel,
        out_shape=jax.ShapeDtypeStruct((1, _NE), jnp.float32),
    )(embedding)
    ni, nj = _M // _BM, _NE // _BN
    idx = pl.pallas_call(
        _dist_argmin_kernel,
        grid=(nj, ni),
        in_specs=[
            pl.BlockSpec((_BM, _D), lambda j, i: (i, 0)),
            pl.BlockSpec((_BN, _D), lambda j, i: (j, 0)),
            pl.BlockSpec((1, _BN), lambda j, i: (0, j)),
        ],
        out_specs=pl.BlockSpec((_BM, 1), lambda j, i: (i, 0)),
        out_shape=jax.ShapeDtypeStruct((_M, 1), jnp.int32),
        scratch_shapes=[
            pltpu.VMEM((_M, 1), jnp.float32),
            pltpu.VMEM((_M, 1), jnp.float32),
            pltpu.VMEM((_M, 1), jnp.int32),
        ],
    )(flat, embedding, cn)
    return idx.reshape(-1)


_NC, _NS = 2, 16
_NW = _NC * _NS
_BPW = _M // _NW


def _sc_gather(table, idx):
    mesh = plsc.VectorSubcoreMesh(core_axis_name="c", subcore_axis_name="s")

    @functools.partial(
        pl.kernel, mesh=mesh,
        out_type=jax.ShapeDtypeStruct((_M, _D), jnp.float32),
        scratch_types=[
            pltpu.VMEM((_BPW,), jnp.int32),
            pltpu.VMEM((_BPW, _D), jnp.float32),
            pltpu.SemaphoreType.DMA,
        ],
    )
    def k(table_hbm, idx_hbm, out_hbm, idx_v, rows_v, sem):
        wid = jax.lax.axis_index("s") * _NC + jax.lax.axis_index("c")
        base = wid * _BPW
        pltpu.sync_copy(idx_hbm.at[pl.ds(base, _BPW)], idx_v)
        pltpu.async_copy(table_hbm.at[idx_v], rows_v, sem).wait()
        pltpu.sync_copy(rows_v, out_hbm.at[pl.ds(base, _BPW)])

    return k(table, idx)


def _ste_loss_kernel(f_ref, q_ref, out_ref, loss_ref, acc_s):
    i = pl.program_id(0)

    @pl.when(i == 0)
    def _():
        acc_s[0] = 0.0

    f = f_ref[...]
    q = q_ref[...].astype(jnp.bfloat16).astype(jnp.float32)
    delta = q - f
    out_ref[...] = f + delta
    acc_s[0] += jnp.sum(delta * delta)
    loss_ref[...] = jnp.full((1, 1), acc_s[0], jnp.float32)


def _ste_and_loss(flat, quantized):
    ni = _M // _BM
    out, loss_sum = pl.pallas_call(
        _ste_loss_kernel,
        grid=(ni,),
        in_specs=[
            pl.BlockSpec((_BM, _D), lambda i: (i, 0)),
            pl.BlockSpec((_BM, _D), lambda i: (i, 0)),
        ],
        out_specs=[
            pl.BlockSpec((_BM, _D), lambda i: (i, 0)),
            pl.BlockSpec((1, 1), lambda i: (0, 0)),
        ],
        out_shape=[
            jax.ShapeDtypeStruct((_M, _D), jnp.float32),
            jax.ShapeDtypeStruct((1, 1), jnp.float32),
        ],
        scratch_shapes=[pltpu.SMEM((1,), jnp.float32)],
    )(flat, quantized)
    mean_sq = loss_sum[0, 0] / (_M * _D)
    loss = mean_sq + _COMMITMENT * mean_sq
    return out, loss


def kernel(inputs, embedding):
    input_shape = inputs.shape
    flat = inputs.reshape(-1, _D)
    encoding_indices = _argmin_codes(flat, embedding)
    quantized = _sc_gather(embedding, encoding_indices)
    quantized_out, loss = _ste_and_loss(flat, quantized)
    return (quantized_out.reshape(input_shape), loss,
            encoding_indices.reshape(input_shape[0], -1))


# speedup vs baseline: 1.1337x; 1.1337x over previous
import functools

import jax
import jax.numpy as jnp
from jax.experimental import pallas as pl
from jax.experimental.pallas import tpu as pltpu
from jax.experimental.pallas import tpu_sc as plsc

_NE = 8192
_D = 256
_M = 8192
_BM = 512
_BN = 4096
_BIG = 2**30
_COMMITMENT = 0.25


def _norms_kernel(e_ref, z_ref, cn_ref, rn_ref):
    e = e_ref[...]
    ones = jnp.ones((1, _D), jnp.float32)
    cn_ref[...] = jax.lax.dot_general(
        ones, e * e, (((1,), (1,)), ((), ())),
        preferred_element_type=jnp.float32)
    z = z_ref[...]
    rn_ref[...] = jnp.sum(z * z, axis=1, keepdims=True)


def _dist_argmin_kernel(z_ref, e_ref, cn_ref, rn_ref, idx_ref, best_s, bidx_s):
    j = pl.program_id(0)
    i = pl.program_id(1)
    row = pl.ds(i * _BM, _BM)

    @pl.when(j == 0)
    def _():
        best_s[row, :] = jnp.full((_BM, 1), jnp.inf, jnp.float32)
        bidx_s[row, :] = jnp.full((_BM, 1), _BIG, jnp.int32)

    @pl.when(j == pl.num_programs(0) // 2)
    def _():
        best_s[row, :] = best_s[row, :].astype(jnp.bfloat16).astype(jnp.float32)

    z = z_ref[...]
    e = e_ref[...]
    mm = jax.lax.dot_general(
        z.astype(jnp.bfloat16), e.astype(jnp.bfloat16),
        (((1,), (1,)), ((), ())),
        preferred_element_type=jnp.float32)
    d = (rn_ref[...] + cn_ref[...]) - 2.0 * mm
    m = jnp.min(d, axis=1, keepdims=True)
    iota = jax.lax.broadcasted_iota(jnp.int32, d.shape, 1)
    i_j = jnp.min(jnp.where(d == m, iota, _BIG), axis=1, keepdims=True) + j * _BN
    upd = m < best_s[row, :]
    bidx_s[row, :] = jnp.where(upd, i_j, bidx_s[row, :])
    best_s[row, :] = jnp.where(upd, m, best_s[row, :])
    idx_ref[...] = bidx_s[row, :]


def _argmin_codes(flat, embedding):
    cn, rn = pl.pallas_call(
        _norms_kernel,
        out_shape=[jax.ShapeDtypeStruct((1, _NE), jnp.float32),
                   jax.ShapeDtypeStruct((_M, 1), jnp.float32)],
    )(embedding, flat)
    ni, nj = _M // _BM, _NE // _BN
    idx = pl.pallas_call(
        _dist_argmin_kernel,
        grid=(nj, ni),
        in_specs=[
            pl.BlockSpec((_BM, _D), lambda j, i: (i, 0)),
            pl.BlockSpec((_BN, _D), lambda j, i: (j, 0)),
            pl.BlockSpec((1, _BN), lambda j, i: (0, j)),
            pl.BlockSpec((_BM, 1), lambda j, i: (i, 0)),
        ],
        out_specs=pl.BlockSpec((_BM, 1), lambda j, i: (i, 0)),
        out_shape=jax.ShapeDtypeStruct((_M, 1), jnp.int32),
        scratch_shapes=[
            pltpu.VMEM((_M, 1), jnp.float32),
            pltpu.VMEM((_M, 1), jnp.int32),
        ],
    )(flat, embedding, cn, rn)
    return idx.reshape(-1)


_NC, _NS = 2, 16
_NW = _NC * _NS
_BPW = _M // _NW


def _sc_gather(table, idx):
    mesh = plsc.VectorSubcoreMesh(core_axis_name="c", subcore_axis_name="s")

    @functools.partial(
        pl.kernel, mesh=mesh,
        out_type=jax.ShapeDtypeStruct((_M, _D), jnp.float32),
        scratch_types=[
            pltpu.VMEM((_BPW,), jnp.int32),
            pltpu.VMEM((_BPW, _D), jnp.float32),
            pltpu.SemaphoreType.DMA,
        ],
    )
    def k(table_hbm, idx_hbm, out_hbm, idx_v, rows_v, sem):
        wid = jax.lax.axis_index("s") * _NC + jax.lax.axis_index("c")
        base = wid * _BPW
        pltpu.sync_copy(idx_hbm.at[pl.ds(base, _BPW)], idx_v)
        pltpu.async_copy(table_hbm.at[idx_v], rows_v, sem).wait()
        pltpu.sync_copy(rows_v, out_hbm.at[pl.ds(base, _BPW)])

    return k(table, idx)


def _ste_loss_kernel(f_ref, q_ref, out_ref, loss_ref, acc_s):
    i = pl.program_id(0)

    @pl.when(i == 0)
    def _():
        acc_s[0] = 0.0

    f = f_ref[...]
    q = q_ref[...].astype(jnp.bfloat16).astype(jnp.float32)
    delta = q - f
    out_ref[...] = f + delta
    acc_s[0] += jnp.sum(delta * delta)
    loss_ref[...] = jnp.full((1, 1), acc_s[0], jnp.float32)


def _ste_and_loss(flat, quantized):
    ni = _M // _BM
    out, loss_sum = pl.pallas_call(
        _ste_loss_kernel,
        grid=(ni,),
        in_specs=[
            pl.BlockSpec((_BM, _D), lambda i: (i, 0)),
            pl.BlockSpec((_BM, _D), lambda i: (i, 0)),
        ],
        out_specs=[
            pl.BlockSpec((_BM, _D), lambda i: (i, 0)),
            pl.BlockSpec((1, 1), lambda i: (0, 0)),
        ],
        out_shape=[
            jax.ShapeDtypeStruct((_M, _D), jnp.float32),
            jax.ShapeDtypeStruct((1, 1), jnp.float32),
        ],
        scratch_shapes=[pltpu.SMEM((1,), jnp.float32)],
    )(flat, quantized)
    mean_sq = loss_sum[0, 0] / (_M * _D)
    loss = mean_sq + _COMMITMENT * mean_sq
    return out, loss


def kernel(inputs, embedding):
    input_shape = inputs.shape
    flat = inputs.reshape(-1, _D)
    encoding_indices = _argmin_codes(flat, embedding)
    quantized = _sc_gather(embedding, encoding_indices)
    quantized_out, loss = _ste_and_loss(flat, quantized)
    return (quantized_out.reshape(input_shape), loss,
            encoding_indices.reshape(input_shape[0], -1))
